# Initial kernel scaffold; baseline (speedup 1.0000x reference)
import jax
import jax.numpy as jnp
from jax.experimental import pallas as pl


def kernel(x_cpg, x_gene, x_mirna, ei_maps_to, ei_targets, ei_ppi, ei_rev_maps_to, ei_rev_targets, Win_cpg, bin_cpg, Win_gene, bin_gene, Win_mirna, bin_mirna, W_l0_maps_to, as_l0_maps_to, ad_l0_maps_to, b_l0_maps_to, W_l0_targets, as_l0_targets, ad_l0_targets, b_l0_targets, W_l0_ppi, as_l0_ppi, ad_l0_ppi, b_l0_ppi, W_l0_rev_maps_to, as_l0_rev_maps_to, ad_l0_rev_maps_to, b_l0_rev_maps_to, W_l0_rev_targets, as_l0_rev_targets, ad_l0_rev_targets, b_l0_rev_targets, lng_l0_cpg, lnb_l0_cpg, lng_l0_gene, lnb_l0_gene, lng_l0_mirna, lnb_l0_mirna, W_l1_maps_to, as_l1_maps_to, ad_l1_maps_to, b_l1_maps_to, W_l1_targets, as_l1_targets, ad_l1_targets, b_l1_targets, W_l1_ppi, as_l1_ppi, ad_l1_ppi, b_l1_ppi, W_l1_rev_maps_to, as_l1_rev_maps_to, ad_l1_rev_maps_to, b_l1_rev_maps_to, W_l1_rev_targets, as_l1_rev_targets, ad_l1_rev_targets, b_l1_rev_targets, lng_l1_cpg, lnb_l1_cpg, lng_l1_gene, lnb_l1_gene, lng_l1_mirna, lnb_l1_mirna):
    raise NotImplementedError("write your pallas kernel here")


# speedup vs baseline: 5.1061x; 5.1061x over previous
import functools

import jax
import jax.numpy as jnp
from jax import lax
from jax.experimental import pallas as pl
from jax.experimental.pallas import tpu as pltpu
from jax.experimental.pallas import tpu_sc as plsc

H = 128
NNODES = {"cpg": 100000, "gene": 20000, "mirna": 2000}
RELSPEC = [("maps_to", "cpg", "gene", False),
           ("targets", "mirna", "gene", False),
           ("ppi", "gene", "gene", True),
           ("rev_maps_to", "gene", "cpg", False),
           ("rev_targets", "gene", "mirna", False)]

_EPS_SM = 1e-16
_EPS_LN = 1e-5


def _rup(x, m):
    return (x + m - 1) // m * m


_BLKP = 512


def _proj_in(x, W, b):
    n = x.shape[0]

    def body(x_ref, w_ref, b_ref, y_ref):
        y_ref[...] = jnp.dot(x_ref[...], w_ref[...],
                             preferred_element_type=jnp.float32) + b_ref[...]

    return pl.pallas_call(
        body,
        grid=(pl.cdiv(n, _BLKP),),
        in_specs=[pl.BlockSpec((_BLKP, H), lambda i: (i, 0)),
                  pl.BlockSpec((H, H), lambda i: (0, 0)),
                  pl.BlockSpec((1, H), lambda i: (0, 0))],
        out_specs=pl.BlockSpec((_BLKP, H), lambda i: (i, 0)),
        out_shape=jax.ShapeDtypeStruct((n, H), jnp.float32),
    )(x, W, b.reshape(1, H))


def _proj_full(x, W, a_s, a_d):
    n = x.shape[0]

    def body(x_ref, w_ref, as_ref, ad_ref, y_ref, als_ref, ald_ref):
        y = jnp.dot(x_ref[...], w_ref[...], preferred_element_type=jnp.float32)
        y_ref[...] = y
        als_ref[...] = jnp.sum(y * as_ref[...], axis=-1)
        ald_ref[...] = jnp.sum(y * ad_ref[...], axis=-1)

    return pl.pallas_call(
        body,
        grid=(pl.cdiv(n, _BLKP),),
        in_specs=[pl.BlockSpec((_BLKP, H), lambda i: (i, 0)),
                  pl.BlockSpec((H, H), lambda i: (0, 0)),
                  pl.BlockSpec((1, H), lambda i: (0, 0)),
                  pl.BlockSpec((1, H), lambda i: (0, 0))],
        out_specs=[pl.BlockSpec((_BLKP, H), lambda i: (i, 0)),
                   pl.BlockSpec((_BLKP,), lambda i: (i,)),
                   pl.BlockSpec((_BLKP,), lambda i: (i,))],
        out_shape=[jax.ShapeDtypeStruct((n, H), jnp.float32),
                   jax.ShapeDtypeStruct((n,), jnp.float32),
                   jax.ShapeDtypeStruct((n,), jnp.float32)],
    )(x, W, a_s.reshape(1, H), a_d.reshape(1, H))


def _proj_al(x, W, a):
    n = x.shape[0]

    def body(x_ref, w_ref, a_ref, al_ref):
        y = jnp.dot(x_ref[...], w_ref[...], preferred_element_type=jnp.float32)
        al_ref[...] = jnp.sum(y * a_ref[...], axis=-1)

    return pl.pallas_call(
        body,
        grid=(pl.cdiv(n, _BLKP),),
        in_specs=[pl.BlockSpec((_BLKP, H), lambda i: (i, 0)),
                  pl.BlockSpec((H, H), lambda i: (0, 0)),
                  pl.BlockSpec((1, H), lambda i: (0, 0))],
        out_specs=pl.BlockSpec((_BLKP,), lambda i: (i,)),
        out_shape=jax.ShapeDtypeStruct((n,), jnp.float32),
    )(x, W, a.reshape(1, H))


def _combine(us, bias_sum, g, bvec, h_prev):
    n = us[0].shape[0]
    k = len(us)
    has_res = h_prev is not None

    def body(*refs):
        u_refs = refs[:k]
        bs_ref, g_ref, b_ref = refs[k:k + 3]
        res_ref = refs[k + 3] if has_res else None
        out_ref = refs[-1]
        acc = u_refs[0][...]
        for r in u_refs[1:]:
            acc = acc + r[...]
        acc = acc + bs_ref[...]
        mu = jnp.mean(acc, axis=-1, keepdims=True)
        var = jnp.mean((acc - mu) ** 2, axis=-1, keepdims=True)
        y = (acc - mu) * lax.rsqrt(var + _EPS_LN) * g_ref[...] + b_ref[...]
        y = jnp.maximum(y, 0.0)
        if has_res:
            y = y + res_ref[...]
        out_ref[...] = y

    specs = [pl.BlockSpec((_BLKP, H), lambda i: (i, 0)) for _ in range(k)]
    specs += [pl.BlockSpec((1, H), lambda i: (0, 0))] * 3
    args = list(us) + [bias_sum.reshape(1, H), g.reshape(1, H), bvec.reshape(1, H)]
    if has_res:
        specs.append(pl.BlockSpec((_BLKP, H), lambda i: (i, 0)))
        args.append(h_prev)
    return pl.pallas_call(
        body,
        grid=(pl.cdiv(n, _BLKP),),
        in_specs=specs,
        out_specs=pl.BlockSpec((_BLKP, H), lambda i: (i, 0)),
        out_shape=jax.ShapeDtypeStruct((n, H), jnp.float32),
    )(*args)


_MESH = plsc.VectorSubcoreMesh(core_axis_name="c", subcore_axis_name="s")
_NC, _NS, _NW = 2, 16, 32
_CHUNK = 8192


@functools.lru_cache(maxsize=None)
def _pass_a_kernel(ep, ndp, e_real):
    per_tile = ep // _NW
    nb = per_tile // 128
    ZB = 2048
    nzc = ndp // ZB

    @functools.partial(
        pl.kernel,
        out_type=(jax.ShapeDtypeStruct((ep,), jnp.float32),
                  jax.ShapeDtypeStruct((2, ndp), jnp.float32)),
        mesh=_MESH,
        scratch_types=[pltpu.VMEM((1, 128), jnp.int32),
                       pltpu.VMEM((1, 128), jnp.int32),
                       pltpu.VMEM((128,), jnp.float32),
                       pltpu.VMEM((128,), jnp.float32),
                       pltpu.VMEM((128,), jnp.float32),
                       pltpu.VMEM((ZB,), jnp.float32),
                       pltpu.VMEM_SHARED((ndp,), jnp.float32),
                       pltpu.SemaphoreType.DMA,
                       pltpu.SemaphoreType.DMA],
    )
    def kern(als_h, ald_h, src_h, dst_h, w_h, s_h,
             src_v, dst_v, asv, adv, wv, zv, s_sh, sem1, sem2):
        c = lax.axis_index("c")
        sid = lax.axis_index("s")
        wid = sid * _NC + c

        def zb(i, _):
            zv[pl.ds(i * 16, 16)] = jnp.zeros((16,), jnp.float32)
            return 0
        lax.fori_loop(0, ZB // 16, zb, 0)

        def zs(ch, _):
            @pl.when(sid == lax.rem(ch, _NS))
            def _():
                pltpu.sync_copy(zv, s_sh.at[pl.ds(ch * ZB, ZB)])
            return 0
        lax.fori_loop(0, nzc, zs, 0)
        plsc.subcore_barrier()

        base = wid * per_tile

        def eb(i, _):
            off = base + i * 128
            pltpu.sync_copy(src_h.at[pl.ds(off, 128)], src_v.at[0])
            pltpu.sync_copy(dst_h.at[pl.ds(off, 128)], dst_v.at[0])
            cp1 = pltpu.async_copy(als_h.at[src_v.at[0]], asv, sem1)
            cp2 = pltpu.async_copy(ald_h.at[dst_v.at[0]], adv, sem2)
            cp1.wait()
            cp2.wait()
            for g in range(8):
                a16 = asv[pl.ds(g * 16, 16)] + adv[pl.ds(g * 16, 16)]
                al = jnp.where(a16 > 0, a16, a16 * jnp.float32(0.2))
                eidx = lax.iota(jnp.int32, 16) + (off + g * 16)
                w16 = jnp.where(eidx < e_real, jnp.exp(al), jnp.float32(0.0))
                wv[pl.ds(g * 16, 16)] = w16
            pltpu.sync_copy(wv, w_h.at[pl.ds(off, 128)])
            pltpu.sync_copy(wv, s_sh.at[dst_v.at[0]], add=True)
            return 0
        lax.fori_loop(0, nb, eb, 0)
        plsc.subcore_barrier()

        @pl.when(sid == 0)
        def _():
            pltpu.sync_copy(s_sh, s_h.at[c])

    return kern


@functools.lru_cache(maxsize=None)
def _pass_b_kernel(ep, ns, ndp):
    per_tile = ep // _NS
    BLK_B = 512
    nb = per_tile // BLK_B
    cap = 784
    chunk_los = list(range(0, ndp, _CHUNK))

    @functools.partial(
        pl.kernel,
        out_type=jax.ShapeDtypeStruct((ndp, H), jnp.float32),
        mesh=_MESH,
        scratch_types=[pltpu.VMEM((BLK_B,), jnp.int32),
                       pltpu.VMEM((BLK_B,), jnp.int32),
                       pltpu.VMEM((BLK_B,), jnp.float32),
                       pltpu.VMEM((cap,), jnp.int32),
                       pltpu.VMEM((cap,), jnp.int32),
                       pltpu.VMEM((cap,), jnp.float32),
                       pltpu.VMEM((1, 128), jnp.int32),
                       pltpu.VMEM((128, H), jnp.float32),
                       pltpu.VMEM((32, H), jnp.float32),
                       pltpu.VMEM((_CHUNK,), jnp.float32),
                       pltpu.VMEM((_CHUNK,), jnp.float32),
                       pltpu.VMEM_SHARED((_CHUNK, H), jnp.float32),
                       pltpu.SemaphoreType.DMA],
        compiler_params=pltpu.CompilerParams(needs_layout_passes=False),
    )
    def kern(ps_h, src_h, dst_h, w_h, s0_h, s1_h, u_h,
             dstb, srcb, wb, cdl, csrc, cw, cidx2, rows, zrows,
             s_loc, s_tmp, u_sh, semg):
        c = lax.axis_index("c")
        sid = lax.axis_index("s")

        z16f = jnp.zeros((16,), jnp.float32)
        z16i = jnp.zeros((16,), jnp.int32)

        def zr(i, _):
            zrows[i // 8, pl.ds((i % 8) * 16, 16)] = z16f
            return 0
        lax.fori_loop(0, 32 * 8, zr, 0)

        def fire(j, _):
            be = j * 128
            for g in range(8):
                cidx2[0, pl.ds(g * 16, 16)] = cdl[pl.ds(be + g * 16, 16)]
            pltpu.async_copy(ps_h.at[csrc.at[pl.ds(be, 128)]],
                             rows, semg).wait()

            def scale(r, _):
                ws = cw[pl.ds(be + r, 16)][0]
                for q in range(8):
                    rows[r, pl.ds(q * 16, 16)] = rows[r, pl.ds(q * 16, 16)] * ws
                return 0
            lax.fori_loop(0, 128, scale, 0)
            pltpu.sync_copy(rows, u_sh.at[cidx2.at[0]], add=True)
            return 0

        for ci, lo in enumerate(chunk_los):
            crows = min(_CHUNK, ndp - lo)

            @pl.when(c == ci % 2)
            def _(lo=lo, crows=crows):
                def zloop(z, _):
                    @pl.when(sid == lax.rem(z, _NS))
                    def _():
                        pltpu.sync_copy(zrows, u_sh.at[pl.ds(z * 32, 32)])
                    return 0
                lax.fori_loop(0, crows // 32, zloop, 0)

                pltpu.sync_copy(s0_h.at[pl.ds(lo, crows)],
                                s_loc.at[pl.ds(0, crows)])
                pltpu.sync_copy(s1_h.at[pl.ds(lo, crows)],
                                s_tmp.at[pl.ds(0, crows)])

                def sadd(i, _):
                    s_loc[pl.ds(i * 16, 16)] = (s_loc[pl.ds(i * 16, 16)]
                                                + s_tmp[pl.ds(i * 16, 16)])
                    return 0
                lax.fori_loop(0, crows // 16, sadd, 0)
                plsc.subcore_barrier()

                def blk(i, cnt):
                    off = sid * per_tile + i * BLK_B
                    pltpu.sync_copy(dst_h.at[pl.ds(off, BLK_B)], dstb)
                    pltpu.sync_copy(src_h.at[pl.ds(off, BLK_B)], srcb)
                    pltpu.sync_copy(w_h.at[pl.ds(off, BLK_B)], wb)

                    def grp(g, cnt):
                        d16 = dstb[pl.ds(g * 16, 16)]
                        s16 = srcb[pl.ds(g * 16, 16)]
                        w16 = wb[pl.ds(g * 16, 16)]
                        m = (d16 >= lo) & (d16 < lo + crows)
                        dl = jnp.clip(d16 - lo, 0, crows - 1)
                        sv = plsc.load_gather(s_loc, [dl])
                        wn = w16 / (sv + jnp.float32(_EPS_SM))
                        plsc.store_compressed(cdl.at[pl.ds(cnt, 16)], dl, mask=m)
                        plsc.store_compressed(csrc.at[pl.ds(cnt, 16)], s16, mask=m)
                        plsc.store_compressed(cw.at[pl.ds(cnt, 16)], wn, mask=m)
                        return cnt + jnp.sum(m.astype(jnp.int32))
                    cnt = lax.fori_loop(0, BLK_B // 16, grp, cnt)

                    ng = cnt // 128
                    lax.fori_loop(0, ng, fire, 0)
                    sh = ng * 128
                    for t in range(8):
                        cdl[pl.ds(t * 16, 16)] = cdl[pl.ds(sh + t * 16, 16)]
                        csrc[pl.ds(t * 16, 16)] = csrc[pl.ds(sh + t * 16, 16)]
                        cw[pl.ds(t * 16, 16)] = cw[pl.ds(sh + t * 16, 16)]
                    return cnt - sh
                cnt = lax.fori_loop(0, nb, blk, jnp.int32(0))

                for t in range(8):
                    cdl[pl.ds(cnt + t * 16, 16)] = z16i
                    csrc[pl.ds(cnt + t * 16, 16)] = z16i
                    cw[pl.ds(cnt + t * 16, 16)] = z16f
                lax.fori_loop(0, (cnt + 127) // 128, fire, 0)
                plsc.subcore_barrier()

                def dloop(z, _):
                    @pl.when(sid == lax.rem(z, _NS))
                    def _():
                        pltpu.sync_copy(u_sh.at[pl.ds(z * 64, 64)],
                                        u_h.at[pl.ds(lo + z * 64, 64)])
                    return 0
                lax.fori_loop(0, crows // 64, dloop, 0)
                plsc.subcore_barrier()

    return kern


def _prep_edges(ei, n_dst, self_loops):
    src = ei[0].astype(jnp.int32)
    dst = ei[1].astype(jnp.int32)
    if self_loops:
        ar = jnp.arange(n_dst, dtype=jnp.int32)
        src = jnp.concatenate([src, ar])
        dst = jnp.concatenate([dst, ar])
    e = src.shape[0]
    ep = _rup(e, 8192)
    src = jnp.pad(src, (0, ep - e))
    dst = jnp.pad(dst, (0, ep - e))
    return src, dst, e, ep


def _forward_impl(p):
    ndp = {t: _rup(n, 2048) for t, n in NNODES.items()}
    h = {t: _proj_in(p["x_" + t], p["Win_" + t], p["bin_" + t]) for t in NNODES}
    edges = {rel: _prep_edges(p["ei_" + rel], NNODES[d], sl)
             for rel, s, d, sl in RELSPEC}

    for l in range(2):
        us = {t: [] for t in NNODES}
        bsum = {t: jnp.zeros((H,), jnp.float32) for t in NNODES}
        for rel, s, d, sl in RELSPEC:
            W = p[f"W_l{l}_{rel}"]
            a_s = p[f"as_l{l}_{rel}"]
            a_d = p[f"ad_l{l}_{rel}"]
            src, dst, e_real, ep = edges[rel]
            if s == d:
                ps, als, ald = _proj_full(h[s], W, a_s, a_d)
            else:
                ps, als, _ = _proj_full(h[s], W, a_s, a_d)
                ald = _proj_al(h[d], W, a_d)
            w_e, s_pair = _pass_a_kernel(ep, ndp[d], e_real)(
                als, ald, src, dst)
            u = _pass_b_kernel(ep, h[s].shape[0], ndp[d])(
                ps, src, dst, w_e, s_pair[0], s_pair[1])
            us[d].append(u[:NNODES[d]])
            bsum[d] = bsum[d] + p[f"b_l{l}_{rel}"]
        hn = {}
        for t in NNODES:
            hn[t] = _combine(us[t], bsum[t], p[f"lng_l{l}_{t}"],
                             p[f"lnb_l{l}_{t}"], h[t] if l > 0 else None)
        h = hn
    return (h["cpg"], h["gene"], h["mirna"])


def kernel(x_cpg, x_gene, x_mirna, ei_maps_to, ei_targets, ei_ppi, ei_rev_maps_to, ei_rev_targets, Win_cpg, bin_cpg, Win_gene, bin_gene, Win_mirna, bin_mirna, W_l0_maps_to, as_l0_maps_to, ad_l0_maps_to, b_l0_maps_to, W_l0_targets, as_l0_targets, ad_l0_targets, b_l0_targets, W_l0_ppi, as_l0_ppi, ad_l0_ppi, b_l0_ppi, W_l0_rev_maps_to, as_l0_rev_maps_to, ad_l0_rev_maps_to, b_l0_rev_maps_to, W_l0_rev_targets, as_l0_rev_targets, ad_l0_rev_targets, b_l0_rev_targets, lng_l0_cpg, lnb_l0_cpg, lng_l0_gene, lnb_l0_gene, lng_l0_mirna, lnb_l0_mirna, W_l1_maps_to, as_l1_maps_to, ad_l1_maps_to, b_l1_maps_to, W_l1_targets, as_l1_targets, ad_l1_targets, b_l1_targets, W_l1_ppi, as_l1_ppi, ad_l1_ppi, b_l1_ppi, W_l1_rev_maps_to, as_l1_rev_maps_to, ad_l1_rev_maps_to, b_l1_rev_maps_to, W_l1_rev_targets, as_l1_rev_targets, ad_l1_rev_targets, b_l1_rev_targets, lng_l1_cpg, lnb_l1_cpg, lng_l1_gene, lnb_l1_gene, lng_l1_mirna, lnb_l1_mirna):
    p = dict(locals())
    return _forward_impl(p)
